# baseline (speedup 1.0000x reference)
import functools

import jax
import jax.numpy as jnp
from jax import lax
from jax.experimental import pallas as pl
from jax.experimental.pallas import tpu as pltpu
from jax.experimental.pallas import tpu_sc as plsc

_NC = 2
_NS = 16
_NW = _NC * _NS
_CHUNK = 128
_PK = 4


def _sc_gather4(upid, ipid, ugp, igp, ump, imp):
    B = upid.shape[0]
    b_per_w = B // _NW
    n_chunks = b_per_w // _CHUNK
    out_t = jax.ShapeDtypeStruct((B, 128), jnp.float32)
    mesh = plsc.VectorSubcoreMesh(core_axis_name="c", subcore_axis_name="s")

    @functools.partial(
        pl.kernel,
        mesh=mesh,
        out_type=[out_t, out_t, out_t, out_t],
        scratch_types=[
            pltpu.VMEM((b_per_w,), jnp.int32),
            pltpu.VMEM((b_per_w,), jnp.int32),
            pltpu.VMEM((_CHUNK, 128), jnp.float32),
            pltpu.VMEM((_CHUNK, 128), jnp.float32),
            pltpu.VMEM((_CHUNK, 128), jnp.float32),
            pltpu.VMEM((_CHUNK, 128), jnp.float32),
            pltpu.SemaphoreType.DMA,
        ],
    )
    def gather4(upid_hbm, ipid_hbm, ug_hbm, ig_hbm, um_hbm, im_hbm,
                ug_o, ig_o, um_o, im_o,
                uidx, iidx, r0, r1, r2, r3, sem):
        wid = lax.axis_index("s") * _NC + lax.axis_index("c")
        base = wid * b_per_w
        pltpu.sync_copy(upid_hbm.at[pl.ds(base, b_per_w)], uidx)
        pltpu.sync_copy(ipid_hbm.at[pl.ds(base, b_per_w)], iidx)
        for j in range(n_chunks):
            sl = pl.ds(j * _CHUNK, _CHUNK)
            copies = [
                pltpu.async_copy(ug_hbm.at[uidx.at[sl]], r0, sem),
                pltpu.async_copy(ig_hbm.at[iidx.at[sl]], r1, sem),
                pltpu.async_copy(um_hbm.at[uidx.at[sl]], r2, sem),
                pltpu.async_copy(im_hbm.at[iidx.at[sl]], r3, sem),
            ]
            for c in copies:
                c.wait()
            out_sl = pl.ds(base + j * _CHUNK, _CHUNK)
            pltpu.sync_copy(r0, ug_o.at[out_sl])
            pltpu.sync_copy(r1, ig_o.at[out_sl])
            pltpu.sync_copy(r2, um_o.at[out_sl])
            pltpu.sync_copy(r3, im_o.at[out_sl])

    return gather4(upid, ipid, ugp, igp, ump, imp)


def _tc_fused(uid, iid, ug, ig, um, im,
              W1u4, W1i4, b1, W2, b2, W3, b3, S, wo_g, wo_h, bo):
    B = ug.shape[0]
    F = S.shape[1]
    TB = 2048
    grid = (B // TB,)

    def body(uid_ref, iid_ref, ug_ref, ig_ref, um_ref, im_ref,
             w1u_ref, w1i_ref, b1_ref, w2_ref, b2_ref, w3_ref, b3_ref,
             s_ref, wog_ref, woh_ref, bo_ref, o_ref):
        lane = lax.broadcasted_iota(jnp.int32, (TB, _PK * F), 1) // F
        umask = (lane == (uid_ref[...] & (_PK - 1))[:, None]).astype(jnp.float32)
        imask = (lane == (iid_ref[...] & (_PK - 1))[:, None]).astype(jnp.float32)
        um_m = um_ref[...] * umask
        im_m = im_ref[...] * imask
        h = jnp.dot(um_m, w1u_ref[...], preferred_element_type=jnp.float32)
        h += jnp.dot(im_m, w1i_ref[...], preferred_element_type=jnp.float32)
        h = jnp.maximum(h + b1_ref[...], 0.0)
        h = jnp.maximum(
            jnp.dot(h, w2_ref[...], preferred_element_type=jnp.float32) + b2_ref[...], 0.0)
        h = jnp.maximum(
            jnp.dot(h, w3_ref[...], preferred_element_type=jnp.float32) + b3_ref[...], 0.0)
        g = (jnp.dot(ug_ref[...] * umask, s_ref[...], preferred_element_type=jnp.float32)
             * jnp.dot(ig_ref[...] * imask, s_ref[...], preferred_element_type=jnp.float32))
        z = (jnp.sum(g * wog_ref[...], axis=1)
             + jnp.sum(h * woh_ref[...], axis=1) + bo_ref[0, 0])
        o_ref[...] = jax.nn.sigmoid(z)

    row_spec = pl.BlockSpec((TB, _PK * F), lambda i: (i, 0))
    id_spec = pl.BlockSpec((TB,), lambda i: (i,))

    def full(a):
        return pl.BlockSpec(a.shape, lambda i: (0,) * a.ndim)

    return pl.pallas_call(
        body,
        grid=grid,
        in_specs=[id_spec, id_spec, row_spec, row_spec, row_spec, row_spec,
                  full(W1u4), full(W1i4), full(b1), full(W2), full(b2),
                  full(W3), full(b3), full(S), full(wo_g), full(wo_h), full(bo)],
        out_specs=pl.BlockSpec((TB,), lambda i: (i,)),
        out_shape=jax.ShapeDtypeStruct((B,), jnp.float32),
    )(uid, iid, ug, ig, um, im,
      W1u4, W1i4, b1, W2, b2, W3, b3, S, wo_g, wo_h, bo)


def kernel(user_id, item_id, user_gmf_w, item_gmf_w, user_mlp_w, item_mlp_w,
           W1, b1, W2, b2, W3, b3, Wo, bo):
    F = user_gmf_w.shape[1]
    uid = user_id.astype(jnp.int32)
    iid = item_id.astype(jnp.int32)
    ug, ig, um, im = _sc_gather4(
        uid >> 2, iid >> 2,
        user_gmf_w.reshape(-1, _PK * F), item_gmf_w.reshape(-1, _PK * F),
        user_mlp_w.reshape(-1, _PK * F), item_mlp_w.reshape(-1, _PK * F))
    W1u4 = jnp.tile(W1[:F], (_PK, 1))
    W1i4 = jnp.tile(W1[F:], (_PK, 1))
    S = jnp.tile(jnp.eye(F, dtype=jnp.float32), (_PK, 1))
    wo_g = Wo[:F].reshape(1, F)
    wo_h = Wo[F:].reshape(1, -1)
    return _tc_fused(uid, iid, ug, ig, um, im,
                     W1u4, W1i4, b1.reshape(1, -1),
                     W2, b2.reshape(1, -1), W3, b3.reshape(1, -1),
                     S, wo_g, wo_h, bo.reshape(1, 1))


# speedup vs baseline: 1.0001x; 1.0001x over previous
import functools

import jax
import jax.numpy as jnp
from jax import lax
from jax.experimental import pallas as pl
from jax.experimental.pallas import tpu as pltpu
from jax.experimental.pallas import tpu_sc as plsc

_NC = 2
_NS = 16
_NW = _NC * _NS
_CHUNK = 128


def _sc_gather4(uid, iid, ug_t, ig_t, um_t, im_t):
    B = uid.shape[0]
    F = ug_t.shape[1]
    b_per_w = B // _NW
    n_chunks = b_per_w // _CHUNK
    out_t = jax.ShapeDtypeStruct((B, F), jnp.float32)
    row_t = pltpu.VMEM((b_per_w, F), jnp.float32)
    mesh = plsc.VectorSubcoreMesh(core_axis_name="c", subcore_axis_name="s")

    @functools.partial(
        pl.kernel,
        mesh=mesh,
        out_type=[out_t, out_t, out_t, out_t],
        scratch_types=[
            pltpu.VMEM((b_per_w,), jnp.int32),
            pltpu.VMEM((b_per_w,), jnp.int32),
            row_t, row_t, row_t, row_t,
            pltpu.SemaphoreType.DMA,
        ],
        compiler_params=pltpu.CompilerParams(use_tc_tiling_on_sc=False),
    )
    def gather4(uid_hbm, iid_hbm, ug_hbm, ig_hbm, um_hbm, im_hbm,
                ug_o, ig_o, um_o, im_o,
                uidx, iidx, r0, r1, r2, r3, sem):
        wid = lax.axis_index("s") * _NC + lax.axis_index("c")
        base = wid * b_per_w
        pltpu.sync_copy(uid_hbm.at[pl.ds(base, b_per_w)], uidx)
        pltpu.sync_copy(iid_hbm.at[pl.ds(base, b_per_w)], iidx)
        copies = []
        for j in range(n_chunks):
            sl = pl.ds(j * _CHUNK, _CHUNK)
            copies += [
                pltpu.async_copy(ug_hbm.at[uidx.at[sl]], r0.at[sl], sem),
                pltpu.async_copy(ig_hbm.at[iidx.at[sl]], r1.at[sl], sem),
                pltpu.async_copy(um_hbm.at[uidx.at[sl]], r2.at[sl], sem),
                pltpu.async_copy(im_hbm.at[iidx.at[sl]], r3.at[sl], sem),
            ]
        for c in copies:
            c.wait()
        out_sl = pl.ds(base, b_per_w)
        pltpu.sync_copy(r0, ug_o.at[out_sl])
        pltpu.sync_copy(r1, ig_o.at[out_sl])
        pltpu.sync_copy(r2, um_o.at[out_sl])
        pltpu.sync_copy(r3, im_o.at[out_sl])

    return gather4(uid, iid, ug_t, ig_t, um_t, im_t)


def _tc_fused(ug, ig, um, im, W1u, W1i, b1, W2, b2, W3, b3, wo_g, wo_h, bo):
    B, F = ug.shape
    TB = 2048
    grid = (B // TB,)

    def body(ug_ref, ig_ref, um_ref, im_ref,
             w1u_ref, w1i_ref, b1_ref, w2_ref, b2_ref, w3_ref, b3_ref,
             wog_ref, woh_ref, bo_ref, o_ref):
        h = jnp.dot(um_ref[...], w1u_ref[...], preferred_element_type=jnp.float32)
        h += jnp.dot(im_ref[...], w1i_ref[...], preferred_element_type=jnp.float32)
        h = jnp.maximum(h + b1_ref[...], 0.0)
        h = jnp.maximum(
            jnp.dot(h, w2_ref[...], preferred_element_type=jnp.float32) + b2_ref[...], 0.0)
        h = jnp.maximum(
            jnp.dot(h, w3_ref[...], preferred_element_type=jnp.float32) + b3_ref[...], 0.0)
        g = ug_ref[...] * ig_ref[...]
        z = (jnp.sum(g * wog_ref[...], axis=1)
             + jnp.sum(h * woh_ref[...], axis=1) + bo_ref[0, 0])
        o_ref[...] = jax.nn.sigmoid(z)

    row_spec = pl.BlockSpec((TB, F), lambda i: (i, 0))

    def full(a):
        return pl.BlockSpec(a.shape, lambda i: (0,) * a.ndim)

    return pl.pallas_call(
        body,
        grid=grid,
        in_specs=[row_spec, row_spec, row_spec, row_spec,
                  full(W1u), full(W1i), full(b1), full(W2), full(b2),
                  full(W3), full(b3), full(wo_g), full(wo_h), full(bo)],
        out_specs=pl.BlockSpec((TB,), lambda i: (i,)),
        out_shape=jax.ShapeDtypeStruct((B,), jnp.float32),
    )(ug, ig, um, im, W1u, W1i, b1, W2, b2, W3, b3, wo_g, wo_h, bo)


def kernel(user_id, item_id, user_gmf_w, item_gmf_w, user_mlp_w, item_mlp_w,
           W1, b1, W2, b2, W3, b3, Wo, bo):
    F = user_gmf_w.shape[1]
    uid = user_id.astype(jnp.int32)
    iid = item_id.astype(jnp.int32)
    ug, ig, um, im = _sc_gather4(
        uid, iid, user_gmf_w, item_gmf_w, user_mlp_w, item_mlp_w)
    return _tc_fused(ug, ig, um, im,
                     W1[:F], W1[F:], b1.reshape(1, -1),
                     W2, b2.reshape(1, -1), W3, b3.reshape(1, -1),
                     Wo[:F].reshape(1, F), Wo[F:].reshape(1, -1),
                     bo.reshape(1, 1))


# speedup vs baseline: 1.4063x; 1.4062x over previous
import functools

import jax
import jax.numpy as jnp
from jax import lax
from jax.experimental import pallas as pl
from jax.experimental.pallas import tpu as pltpu
from jax.experimental.pallas import tpu_sc as plsc

_NC = 2
_NS = 16
_NW = _NC * _NS
_CH = 64
_VL = 16


def _sc_gather4(uid, iid, ug_t, ig_t, um_t, im_t):
    B = uid.shape[0]
    F = ug_t.shape[1]
    b_per_w = B // _NW
    n_rounds = b_per_w // _CH
    out_t = jax.ShapeDtypeStruct((B, F), jnp.float32)
    row_t = pltpu.VMEM((_CH, F), jnp.float32)
    mesh = plsc.VectorSubcoreMesh(core_axis_name="c", subcore_axis_name="s")

    @functools.partial(
        pl.kernel,
        mesh=mesh,
        out_type=[out_t, out_t, out_t, out_t],
        scratch_types=[
            pltpu.VMEM((b_per_w,), jnp.int32),
            pltpu.VMEM((b_per_w,), jnp.int32),
            row_t, row_t, row_t, row_t,
            pltpu.SemaphoreType.DMA,
        ],
        compiler_params=pltpu.CompilerParams(needs_layout_passes=False),
    )
    def gather4(uid_hbm, iid_hbm, ug_hbm, ig_hbm, um_hbm, im_hbm,
                ug_o, ig_o, um_o, im_o,
                uidx, iidx, r0, r1, r2, r3, sem):
        wid = lax.axis_index("s") * _NC + lax.axis_index("c")
        base = wid * b_per_w
        pltpu.sync_copy(uid_hbm.at[pl.ds(base, b_per_w)], uidx)
        pltpu.sync_copy(iid_hbm.at[pl.ds(base, b_per_w)], iidx)
        lane = lax.broadcasted_iota(jnp.int32, (_VL,), 0)

        def round_body(r, carry):
            off = r * _CH
            for c in range(_CH // _VL):
                uv = uidx[pl.ds(off + c * _VL, _VL)]
                iv = iidx[pl.ds(off + c * _VL, _VL)]
                for j in range(_VL):
                    us = jnp.max(jnp.where(lane == j, uv, 0))
                    vs = jnp.max(jnp.where(lane == j, iv, 0))
                    k = c * _VL + j
                    dst = pl.ds(k, 1)
                    pltpu.async_copy(ug_hbm.at[pl.ds(us, 1)], r0.at[dst], sem)
                    pltpu.async_copy(ig_hbm.at[pl.ds(vs, 1)], r1.at[dst], sem)
                    pltpu.async_copy(um_hbm.at[pl.ds(us, 1)], r2.at[dst], sem)
                    pltpu.async_copy(im_hbm.at[pl.ds(vs, 1)], r3.at[dst], sem)
            for buf in (r0, r1, r2, r3):
                pltpu.make_async_copy(ug_hbm.at[pl.ds(0, _CH)], buf, sem).wait()
            out_sl = pl.ds(base + off, _CH)
            pltpu.sync_copy(r0, ug_o.at[out_sl])
            pltpu.sync_copy(r1, ig_o.at[out_sl])
            pltpu.sync_copy(r2, um_o.at[out_sl])
            pltpu.sync_copy(r3, im_o.at[out_sl])
            return carry

        lax.fori_loop(0, n_rounds, round_body, 0)

    return gather4(uid, iid, ug_t, ig_t, um_t, im_t)


def _tc_fused(ug, ig, um, im, W1u, W1i, b1, W2, b2, W3, b3, wo_g, wo_h, bo):
    B, F = ug.shape
    TB = 2048
    grid = (B // TB,)

    def body(ug_ref, ig_ref, um_ref, im_ref,
             w1u_ref, w1i_ref, b1_ref, w2_ref, b2_ref, w3_ref, b3_ref,
             wog_ref, woh_ref, bo_ref, o_ref):
        h = jnp.dot(um_ref[...], w1u_ref[...], preferred_element_type=jnp.float32)
        h += jnp.dot(im_ref[...], w1i_ref[...], preferred_element_type=jnp.float32)
        h = jnp.maximum(h + b1_ref[...], 0.0)
        h = jnp.maximum(
            jnp.dot(h, w2_ref[...], preferred_element_type=jnp.float32) + b2_ref[...], 0.0)
        h = jnp.maximum(
            jnp.dot(h, w3_ref[...], preferred_element_type=jnp.float32) + b3_ref[...], 0.0)
        g = ug_ref[...] * ig_ref[...]
        z = (jnp.sum(g * wog_ref[...], axis=1)
             + jnp.sum(h * woh_ref[...], axis=1) + bo_ref[0, 0])
        o_ref[...] = jax.nn.sigmoid(z)

    row_spec = pl.BlockSpec((TB, F), lambda i: (i, 0))

    def full(a):
        return pl.BlockSpec(a.shape, lambda i: (0,) * a.ndim)

    return pl.pallas_call(
        body,
        grid=grid,
        in_specs=[row_spec, row_spec, row_spec, row_spec,
                  full(W1u), full(W1i), full(b1), full(W2), full(b2),
                  full(W3), full(b3), full(wo_g), full(wo_h), full(bo)],
        out_specs=pl.BlockSpec((TB,), lambda i: (i,)),
        out_shape=jax.ShapeDtypeStruct((B,), jnp.float32),
    )(ug, ig, um, im, W1u, W1i, b1, W2, b2, W3, b3, wo_g, wo_h, bo)


def kernel(user_id, item_id, user_gmf_w, item_gmf_w, user_mlp_w, item_mlp_w,
           W1, b1, W2, b2, W3, b3, Wo, bo):
    F = user_gmf_w.shape[1]
    uid = user_id.astype(jnp.int32)
    iid = item_id.astype(jnp.int32)
    ug, ig, um, im = _sc_gather4(
        uid, iid, user_gmf_w, item_gmf_w, user_mlp_w, item_mlp_w)
    return _tc_fused(ug, ig, um, im,
                     W1[:F], W1[F:], b1.reshape(1, -1),
                     W2, b2.reshape(1, -1), W3, b3.reshape(1, -1),
                     Wo[:F].reshape(1, F), Wo[F:].reshape(1, -1),
                     bo.reshape(1, 1))


# speedup vs baseline: 1.4064x; 1.0001x over previous
import functools

import jax
import jax.numpy as jnp
from jax import lax
from jax.experimental import pallas as pl
from jax.experimental.pallas import tpu as pltpu
from jax.experimental.pallas import tpu_sc as plsc

_NC = 2
_NS = 16
_NW = _NC * _NS
_CH = 64
_VL = 16


def _sc_gather4(uid, iid, ug_t, ig_t, um_t, im_t):
    B = uid.shape[0]
    F = ug_t.shape[1]
    b_per_w = B // _NW
    n_rounds = b_per_w // _CH
    out_t = jax.ShapeDtypeStruct((B, F), jnp.float32)
    row_t = pltpu.VMEM((_CH, F), jnp.float32)
    mesh = plsc.VectorSubcoreMesh(core_axis_name="c", subcore_axis_name="s")

    @functools.partial(
        pl.kernel,
        mesh=mesh,
        out_type=[out_t, out_t, out_t, out_t],
        scratch_types=[
            pltpu.VMEM((b_per_w,), jnp.int32),
            pltpu.VMEM((b_per_w,), jnp.int32),
            row_t, row_t, row_t, row_t,
            pltpu.SemaphoreType.DMA,
        ],
    )
    def gather4(uid_hbm, iid_hbm, ug_hbm, ig_hbm, um_hbm, im_hbm,
                ug_o, ig_o, um_o, im_o,
                uidx, iidx, r0, r1, r2, r3, sem):
        wid = lax.axis_index("s") * _NC + lax.axis_index("c")
        base = wid * b_per_w
        pltpu.sync_copy(uid_hbm.at[pl.ds(base, b_per_w)], uidx)
        pltpu.sync_copy(iid_hbm.at[pl.ds(base, b_per_w)], iidx)

        def round_body(r, carry):
            off = r * _CH
            for c in range(_CH // _VL):
                uv = uidx[pl.ds(off + c * _VL, _VL)]
                iv = iidx[pl.ds(off + c * _VL, _VL)]
                for j in range(_VL):
                    us = uv[j]
                    vs = iv[j]
                    k = c * _VL + j
                    dst = pl.ds(k, 1)
                    pltpu.async_copy(ug_hbm.at[pl.ds(us, 1)], r0.at[dst], sem)
                    pltpu.async_copy(ig_hbm.at[pl.ds(vs, 1)], r1.at[dst], sem)
                    pltpu.async_copy(um_hbm.at[pl.ds(us, 1)], r2.at[dst], sem)
                    pltpu.async_copy(im_hbm.at[pl.ds(vs, 1)], r3.at[dst], sem)
            for buf in (r0, r1, r2, r3):
                pltpu.make_async_copy(ug_hbm.at[pl.ds(0, _CH)], buf, sem).wait()
            out_sl = pl.ds(base + off, _CH)
            pltpu.sync_copy(r0, ug_o.at[out_sl])
            pltpu.sync_copy(r1, ig_o.at[out_sl])
            pltpu.sync_copy(r2, um_o.at[out_sl])
            pltpu.sync_copy(r3, im_o.at[out_sl])
            return carry

        lax.fori_loop(0, n_rounds, round_body, 0)

    return gather4(uid, iid, ug_t, ig_t, um_t, im_t)


def _tc_fused(ug, ig, um, im, W1u, W1i, b1, W2, b2, W3, b3, wo_g, wo_h, bo):
    B, F = ug.shape
    TB = 2048
    grid = (B // TB,)

    def body(ug_ref, ig_ref, um_ref, im_ref,
             w1u_ref, w1i_ref, b1_ref, w2_ref, b2_ref, w3_ref, b3_ref,
             wog_ref, woh_ref, bo_ref, o_ref):
        h = jnp.dot(um_ref[...], w1u_ref[...], preferred_element_type=jnp.float32)
        h += jnp.dot(im_ref[...], w1i_ref[...], preferred_element_type=jnp.float32)
        h = jnp.maximum(h + b1_ref[...], 0.0)
        h = jnp.maximum(
            jnp.dot(h, w2_ref[...], preferred_element_type=jnp.float32) + b2_ref[...], 0.0)
        h = jnp.maximum(
            jnp.dot(h, w3_ref[...], preferred_element_type=jnp.float32) + b3_ref[...], 0.0)
        g = ug_ref[...] * ig_ref[...]
        z = (jnp.sum(g * wog_ref[...], axis=1)
             + jnp.sum(h * woh_ref[...], axis=1) + bo_ref[0, 0])
        o_ref[...] = jax.nn.sigmoid(z)

    row_spec = pl.BlockSpec((TB, F), lambda i: (i, 0))

    def full(a):
        return pl.BlockSpec(a.shape, lambda i: (0,) * a.ndim)

    return pl.pallas_call(
        body,
        grid=grid,
        in_specs=[row_spec, row_spec, row_spec, row_spec,
                  full(W1u), full(W1i), full(b1), full(W2), full(b2),
                  full(W3), full(b3), full(wo_g), full(wo_h), full(bo)],
        out_specs=pl.BlockSpec((TB,), lambda i: (i,)),
        out_shape=jax.ShapeDtypeStruct((B,), jnp.float32),
    )(ug, ig, um, im, W1u, W1i, b1, W2, b2, W3, b3, wo_g, wo_h, bo)


def kernel(user_id, item_id, user_gmf_w, item_gmf_w, user_mlp_w, item_mlp_w,
           W1, b1, W2, b2, W3, b3, Wo, bo):
    F = user_gmf_w.shape[1]
    uid = user_id.astype(jnp.int32)
    iid = item_id.astype(jnp.int32)
    ug, ig, um, im = _sc_gather4(
        uid, iid, user_gmf_w, item_gmf_w, user_mlp_w, item_mlp_w)
    return _tc_fused(ug, ig, um, im,
                     W1[:F], W1[F:], b1.reshape(1, -1),
                     W2, b2.reshape(1, -1), W3, b3.reshape(1, -1),
                     Wo[:F].reshape(1, F), Wo[F:].reshape(1, -1),
                     bo.reshape(1, 1))
